# Initial kernel scaffold; baseline (speedup 1.0000x reference)
import functools

import jax
import jax.numpy as jnp
from jax import lax
from jax.experimental import pallas as pl
from jax.experimental.pallas import tpu as pltpu
from jax.experimental.pallas import tpu_sc as plsc

_N = 10000
_E = 320000
_D_IN = 128
_H = 256
_D_OUT = 128
_DEPTH = 3

_NC = 2
_NS = 16
_B = 128

_EPAD = 323584
_EROWS = _EPAD // _B
_RPW_FEAT = _EROWS // _NS
_RPW_HIST = _EROWS // (_NC * _NS)
_NTAB = 10016
_TPW = _NTAB // _NS

_f32 = jnp.float32
_i32 = jnp.int32

_MESH = plsc.VectorSubcoreMesh(core_axis_name="c", subcore_axis_name="s",
                               num_cores=_NC, num_subcores=_NS)


def _fill(buf, nrows, ncols, value):
    @pl.loop(0, nrows)
    def _(r):
        @pl.loop(0, ncols // 16)
        def _(j):
            buf[r, pl.ds(j * 16, 16)] = jnp.full((16,), value, _f32)


def _init_table_slice(zero_src, table, base, nrows):
    full, rem = nrows // _B, nrows % _B
    for k in range(full):
        pltpu.sync_copy(zero_src, table.at[pl.ds(base + k * _B, _B)])
    if rem:
        pltpu.sync_copy(zero_src.at[pl.ds(0, rem)],
                        table.at[pl.ds(base + full * _B, rem)])


@functools.partial(
    pl.kernel,
    out_type=(jax.ShapeDtypeStruct((_NTAB, 16), _f32),
              jax.ShapeDtypeStruct((_NTAB, 16), _f32)),
    mesh=_MESH,
    scratch_types=[
        pltpu.VMEM((_RPW_HIST, _B), _i32),
        pltpu.VMEM((_B, 16), _f32),
        pltpu.VMEM_SHARED((_NTAB, 16), _f32),
    ],
)
def _sc_degree(dst_hbm, out0, out1, dstv, buf, table):
    c = lax.axis_index("c")
    s = lax.axis_index("s")
    w = c * _NS + s

    _fill(buf, _B, 16, 0.0)
    _init_table_slice(buf, table, s * _TPW, _TPW)
    _fill(buf, _B, 16, 1.0)
    plsc.subcore_barrier()

    pltpu.sync_copy(dst_hbm.at[pl.ds(w * _RPW_HIST, _RPW_HIST)], dstv)

    @pl.loop(0, _RPW_HIST)
    def _(b):
        pltpu.sync_copy(buf, table.at[dstv.at[b]], add=True)

    plsc.subcore_barrier()

    @pl.when(c == 0)
    def _():
        pltpu.sync_copy(table.at[pl.ds(s * _TPW, _TPW)],
                        out0.at[pl.ds(s * _TPW, _TPW)])

    @pl.when(c == 1)
    def _():
        pltpu.sync_copy(table.at[pl.ds(s * _TPW, _TPW)],
                        out1.at[pl.ds(s * _TPW, _TPW)])


@functools.partial(
    pl.kernel,
    out_type=(jax.ShapeDtypeStruct((_NTAB, 128), _f32),
              jax.ShapeDtypeStruct((_NTAB, 128), _f32)),
    mesh=_MESH,
    scratch_types=[
        pltpu.VMEM((_RPW_FEAT, _B), _i32),
        pltpu.VMEM((_RPW_FEAT, _B), _i32),
        pltpu.VMEM((_B, 128), _f32),
        pltpu.VMEM_SHARED((_NTAB, 128), _f32),
        pltpu.SemaphoreType.DMA,
    ],
)
def _sc_aggregate(hlo_hbm, hhi_hbm, src_hbm, dst_hbm, alo, ahi,
                  srcv, dstv, rows, table, sem):
    c = lax.axis_index("c")
    s = lax.axis_index("s")

    _fill(rows, _B, 128, 0.0)
    _init_table_slice(rows, table, s * _TPW, _TPW)
    plsc.subcore_barrier()

    pltpu.sync_copy(src_hbm.at[pl.ds(s * _RPW_FEAT, _RPW_FEAT)], srcv)
    pltpu.sync_copy(dst_hbm.at[pl.ds(s * _RPW_FEAT, _RPW_FEAT)], dstv)

    @pl.loop(0, _RPW_FEAT)
    def _(b):
        @pl.when(c == 0)
        def _():
            pltpu.async_copy(hlo_hbm.at[srcv.at[b]], rows, sem).wait()

        @pl.when(c == 1)
        def _():
            pltpu.async_copy(hhi_hbm.at[srcv.at[b]], rows, sem).wait()

        pltpu.sync_copy(rows, table.at[dstv.at[b]], add=True)

    plsc.subcore_barrier()

    @pl.when(c == 0)
    def _():
        pltpu.sync_copy(table.at[pl.ds(s * _TPW, _TPW)],
                        alo.at[pl.ds(s * _TPW, _TPW)])

    @pl.when(c == 1)
    def _():
        pltpu.sync_copy(table.at[pl.ds(s * _TPW, _TPW)],
                        ahi.at[pl.ds(s * _TPW, _TPW)])


_BLK = 1000


def _dot(a, b):
    return lax.dot_general(a, b, (((1,), (0,)), ((), ())),
                           precision=lax.Precision.HIGHEST,
                           preferred_element_type=_f32)


def _dinv_block(h0_ref, h1_ref):
    deg = 1.0 + h0_ref[...][:, 0:1] + h1_ref[...][:, 0:1]
    return lax.rsqrt(deg)


def _encoder_body(x_ref, w1_ref, b1_ref, w2_ref, b2_ref, o_ref):
    h = jnp.maximum(_dot(x_ref[...], w1_ref[...]) + b1_ref[...], 0.0)
    o_ref[...] = _dot(h, w2_ref[...]) + b2_ref[...]


def _prescale_body(h_ref, h0_ref, h1_ref, lo_ref, hi_ref):
    dinv = _dinv_block(h0_ref, h1_ref)
    hs = dinv * h_ref[...]
    lo_ref[...] = hs[:, :128]
    hi_ref[...] = hs[:, 128:]


def _conv_mid_body(alo_ref, ahi_ref, h_ref, h0_ref, h1_ref, w_ref, b_ref,
                   oh_ref, lo_ref, hi_ref):
    dinv = _dinv_block(h0_ref, h1_ref)
    agg = jnp.concatenate([alo_ref[...], ahi_ref[...]], axis=1)
    a = dinv * agg + (dinv * dinv) * h_ref[...]
    y = _dot(a, w_ref[...]) + b_ref[...]
    hn = h_ref[...] + jnp.maximum(y, 0.0)
    oh_ref[...] = hn
    hs = dinv * hn
    lo_ref[...] = hs[:, :128]
    hi_ref[...] = hs[:, 128:]


def _conv_last_body(alo_ref, ahi_ref, h_ref, h0_ref, h1_ref, w_ref, b_ref,
                    dw1_ref, db1_ref, dw2_ref, db2_ref, y_ref):
    dinv = _dinv_block(h0_ref, h1_ref)
    agg = jnp.concatenate([alo_ref[...], ahi_ref[...]], axis=1)
    a = dinv * agg + (dinv * dinv) * h_ref[...]
    y = _dot(a, w_ref[...]) + b_ref[...]
    hn = h_ref[...] + jnp.maximum(y, 0.0)
    y1 = jnp.maximum(_dot(hn, dw1_ref[...]) + db1_ref[...], 0.0)
    y_ref[...] = _dot(y1, dw2_ref[...]) + db2_ref[...]


def _row_spec(cols):
    return pl.BlockSpec((_BLK, cols), lambda i: (i, 0))


def _full_spec(shape):
    nd = len(shape)
    return pl.BlockSpec(shape, lambda i: (0,) * nd)


def _tc_call(body, in_arrays, row_cols_in, row_cols_out):
    in_specs = [_row_spec(cc) for cc in row_cols_in]
    in_specs += [_full_spec(a.shape) for a in in_arrays[len(row_cols_in):]]
    out_specs = tuple(_row_spec(cc) for cc in row_cols_out)
    out_shape = tuple(jax.ShapeDtypeStruct((_N, cc), _f32)
                      for cc in row_cols_out)
    if len(out_shape) == 1:
        out_shape, out_specs = out_shape[0], out_specs[0]
    return pl.pallas_call(
        body,
        grid=(_N // _BLK,),
        in_specs=in_specs,
        out_specs=out_specs,
        out_shape=out_shape,
    )(*in_arrays)


def kernel(x, edge_index, enc_W1, enc_b1, enc_W2, enc_b2, conv_W, conv_b,
           dec_W1, dec_b1, dec_W2, dec_b2):
    pad = _EPAD - _E
    src = jnp.concatenate([edge_index[0], jnp.zeros((pad,), _i32)])
    dst = jnp.concatenate([edge_index[1], jnp.full((pad,), _N, _i32)])
    src2d = src.reshape(_EROWS, _B)
    dst2d = dst.reshape(_EROWS, _B)

    hist0, hist1 = _sc_degree(dst2d)
    h0 = hist0[:_N]
    h1 = hist1[:_N]

    h = _tc_call(_encoder_body,
                 [x, enc_W1, enc_b1.reshape(1, _H), enc_W2,
                  enc_b2.reshape(1, _H)],
                 [_D_IN], [_H])

    hlo, hhi = _tc_call(_prescale_body, [h, h0, h1],
                        [_H, 16, 16], [128, 128])

    for l in range(_DEPTH):
        alo, ahi = _sc_aggregate(hlo, hhi, src2d, dst2d)
        alo = alo[:_N]
        ahi = ahi[:_N]
        if l < _DEPTH - 1:
            h, hlo, hhi = _tc_call(
                _conv_mid_body,
                [alo, ahi, h, h0, h1, conv_W[l], conv_b[l].reshape(1, _H)],
                [128, 128, _H, 16, 16], [_H, 128, 128])
        else:
            y = _tc_call(
                _conv_last_body,
                [alo, ahi, h, h0, h1, conv_W[l], conv_b[l].reshape(1, _H),
                 dec_W1, dec_b1.reshape(1, _H), dec_W2,
                 dec_b2.reshape(1, _D_OUT)],
                [128, 128, _H, 16, 16], [_D_OUT])
    return y


# speedup vs baseline: 6.7080x; 6.7080x over previous
import functools

import jax
import jax.numpy as jnp
from jax import lax
from jax.experimental import pallas as pl
from jax.experimental.pallas import tpu as pltpu
from jax.experimental.pallas import tpu_sc as plsc

_N = 10000
_E = 320000
_D_IN = 128
_H = 256
_D_OUT = 128
_DEPTH = 3

_NC = 2
_NS = 16
_B = 128

_EPAD = 327680
_EROWS = _EPAD // _B
_RPW_FEAT = _EROWS // _NS
_RPW_HIST = _EROWS // (_NC * _NS)
_NTAB = 10240
_TPW = _NTAB // _NS
_CHUNK = 16

_f32 = jnp.float32
_i32 = jnp.int32

@functools.cache
def _mesh():
    return plsc.VectorSubcoreMesh(core_axis_name="c", subcore_axis_name="s",
                                  num_cores=_NC, num_subcores=_NS)


def _fill(buf, nrows, ncols, value):
    @pl.loop(0, nrows)
    def _(r):
        @pl.loop(0, ncols // 16)
        def _(j):
            buf[r, pl.ds(j * 16, 16)] = jnp.full((16,), value, _f32)


def _init_table_slice(zero_src, table, base, nrows):
    full, rem = nrows // _B, nrows % _B
    for k in range(full):
        pltpu.sync_copy(zero_src, table.at[pl.ds(base + k * _B, _B)])
    if rem:
        pltpu.sync_copy(zero_src.at[pl.ds(0, rem)],
                        table.at[pl.ds(base + full * _B, rem)])


@functools.cache
def _sc_degree_kernel():
    return pl.kernel(
        _sc_degree_body,
        out_type=(jax.ShapeDtypeStruct((_NTAB, 16), _f32),
                  jax.ShapeDtypeStruct((_NTAB, 16), _f32)),
        mesh=_mesh(),
        scratch_types=[
            pltpu.VMEM((_RPW_HIST, _B), _i32),
            pltpu.VMEM((_B, 16), _f32),
            pltpu.VMEM_SHARED((_NTAB, 16), _f32),
        ],
    )


def _sc_degree_body(dst_hbm, out0, out1, dstv, buf, table):
    c = lax.axis_index("c")
    s = lax.axis_index("s")
    w = c * _NS + s

    _fill(buf, _B, 16, 0.0)
    _init_table_slice(buf, table, s * _TPW, _TPW)
    _fill(buf, _B, 16, 1.0)
    plsc.subcore_barrier()

    pltpu.sync_copy(dst_hbm.at[pl.ds(w * _RPW_HIST, _RPW_HIST)], dstv)

    @pl.loop(0, _RPW_HIST)
    def _(b):
        pltpu.sync_copy(buf, table.at[dstv.at[b]], add=True)

    plsc.subcore_barrier()

    @pl.when(c == 0)
    def _():
        pltpu.sync_copy(table.at[pl.ds(s * _TPW, _TPW)],
                        out0.at[pl.ds(s * _TPW, _TPW)])

    @pl.when(c == 1)
    def _():
        pltpu.sync_copy(table.at[pl.ds(s * _TPW, _TPW)],
                        out1.at[pl.ds(s * _TPW, _TPW)])


@functools.cache
def _sc_aggregate_kernel():
    return pl.kernel(
        _sc_aggregate_body,
        out_type=(jax.ShapeDtypeStruct((_NTAB, 128), _f32),
                  jax.ShapeDtypeStruct((_NTAB, 128), _f32)),
        mesh=_mesh(),
        scratch_types=[
            pltpu.VMEM((_CHUNK, _B), _i32),
            pltpu.VMEM((_CHUNK, _B), _i32),
            pltpu.VMEM((_B, 128), _f32),
            pltpu.VMEM_SHARED((_NTAB, 128), _f32),
            pltpu.SemaphoreType.DMA,
        ],
    )


def _sc_aggregate_body(hlo_hbm, hhi_hbm, src_hbm, dst_hbm, alo, ahi,
                       srcv, dstv, rows, table, sem):
    c = lax.axis_index("c")
    s = lax.axis_index("s")

    _fill(rows, _B, 128, 0.0)
    _init_table_slice(rows, table, s * _TPW, _TPW)
    plsc.subcore_barrier()

    @pl.loop(0, _RPW_FEAT // _CHUNK)
    def _(ck):
        base = s * _RPW_FEAT + ck * _CHUNK
        pltpu.sync_copy(src_hbm.at[pl.ds(base, _CHUNK)], srcv)
        pltpu.sync_copy(dst_hbm.at[pl.ds(base, _CHUNK)], dstv)

        @pl.loop(0, _CHUNK)
        def _(b):
            @pl.when(c == 0)
            def _():
                pltpu.async_copy(hlo_hbm.at[srcv.at[b]], rows, sem).wait()

            @pl.when(c == 1)
            def _():
                pltpu.async_copy(hhi_hbm.at[srcv.at[b]], rows, sem).wait()

            pltpu.sync_copy(rows, table.at[dstv.at[b]], add=True)

    plsc.subcore_barrier()

    @pl.when(c == 0)
    def _():
        pltpu.sync_copy(table.at[pl.ds(s * _TPW, _TPW)],
                        alo.at[pl.ds(s * _TPW, _TPW)])

    @pl.when(c == 1)
    def _():
        pltpu.sync_copy(table.at[pl.ds(s * _TPW, _TPW)],
                        ahi.at[pl.ds(s * _TPW, _TPW)])


_BLK = 1000


def _dot(a, b):
    return lax.dot_general(a, b, (((1,), (0,)), ((), ())),
                           precision=lax.Precision.HIGHEST,
                           preferred_element_type=_f32)


def _dinv_block(h0_ref, h1_ref):
    deg = 1.0 + h0_ref[...][:, 0:1] + h1_ref[...][:, 0:1]
    return lax.rsqrt(deg)


def _encoder_body(x_ref, w1_ref, b1_ref, w2_ref, b2_ref, o_ref):
    h = jnp.maximum(_dot(x_ref[...], w1_ref[...]) + b1_ref[...], 0.0)
    o_ref[...] = _dot(h, w2_ref[...]) + b2_ref[...]


def _prescale_body(h_ref, h0_ref, h1_ref, lo_ref, hi_ref):
    dinv = _dinv_block(h0_ref, h1_ref)
    hs = dinv * h_ref[...]
    lo_ref[...] = hs[:, :128]
    hi_ref[...] = hs[:, 128:]


def _conv_mid_body(alo_ref, ahi_ref, h_ref, h0_ref, h1_ref, w_ref, b_ref,
                   oh_ref, lo_ref, hi_ref):
    dinv = _dinv_block(h0_ref, h1_ref)
    agg = jnp.concatenate([alo_ref[...], ahi_ref[...]], axis=1)
    a = dinv * agg + (dinv * dinv) * h_ref[...]
    y = _dot(a, w_ref[...]) + b_ref[...]
    hn = h_ref[...] + jnp.maximum(y, 0.0)
    oh_ref[...] = hn
    hs = dinv * hn
    lo_ref[...] = hs[:, :128]
    hi_ref[...] = hs[:, 128:]


def _conv_last_body(alo_ref, ahi_ref, h_ref, h0_ref, h1_ref, w_ref, b_ref,
                    dw1_ref, db1_ref, dw2_ref, db2_ref, y_ref):
    dinv = _dinv_block(h0_ref, h1_ref)
    agg = jnp.concatenate([alo_ref[...], ahi_ref[...]], axis=1)
    a = dinv * agg + (dinv * dinv) * h_ref[...]
    y = _dot(a, w_ref[...]) + b_ref[...]
    hn = h_ref[...] + jnp.maximum(y, 0.0)
    y1 = jnp.maximum(_dot(hn, dw1_ref[...]) + db1_ref[...], 0.0)
    y_ref[...] = _dot(y1, dw2_ref[...]) + db2_ref[...]


def _row_spec(cols):
    return pl.BlockSpec((_BLK, cols), lambda i: (i, 0))


def _full_spec(shape):
    nd = len(shape)
    return pl.BlockSpec(shape, lambda i: (0,) * nd)


def _tc_call(body, in_arrays, row_cols_in, row_cols_out):
    in_specs = [_row_spec(cc) for cc in row_cols_in]
    in_specs += [_full_spec(a.shape) for a in in_arrays[len(row_cols_in):]]
    out_specs = tuple(_row_spec(cc) for cc in row_cols_out)
    out_shape = tuple(jax.ShapeDtypeStruct((_N, cc), _f32)
                      for cc in row_cols_out)
    if len(out_shape) == 1:
        out_shape, out_specs = out_shape[0], out_specs[0]
    return pl.pallas_call(
        body,
        grid=(_N // _BLK,),
        in_specs=in_specs,
        out_specs=out_specs,
        out_shape=out_shape,
    )(*in_arrays)


def kernel(x, edge_index, enc_W1, enc_b1, enc_W2, enc_b2, conv_W, conv_b,
           dec_W1, dec_b1, dec_W2, dec_b2):
    pad = _EPAD - _E
    src = jnp.concatenate([edge_index[0], jnp.zeros((pad,), _i32)])
    dst = jnp.concatenate([edge_index[1], jnp.full((pad,), _N, _i32)])
    src2d = src.reshape(_EROWS, _B)
    dst2d = dst.reshape(_EROWS, _B)

    hist0, hist1 = _sc_degree_kernel()(dst2d)
    h0 = hist0[:_N]
    h1 = hist1[:_N]

    h = _tc_call(_encoder_body,
                 [x, enc_W1, enc_b1.reshape(1, _H), enc_W2,
                  enc_b2.reshape(1, _H)],
                 [_D_IN], [_H])

    hlo, hhi = _tc_call(_prescale_body, [h, h0, h1],
                        [_H, 16, 16], [128, 128])

    for l in range(_DEPTH):
        alo, ahi = _sc_aggregate_kernel()(hlo, hhi, src2d, dst2d)
        alo = alo[:_N]
        ahi = ahi[:_N]
        if l < _DEPTH - 1:
            h, hlo, hhi = _tc_call(
                _conv_mid_body,
                [alo, ahi, h, h0, h1, conv_W[l], conv_b[l].reshape(1, _H)],
                [128, 128, _H, 16, 16], [_H, 128, 128])
        else:
            y = _tc_call(
                _conv_last_body,
                [alo, ahi, h, h0, h1, conv_W[l], conv_b[l].reshape(1, _H),
                 dec_W1, dec_b1.reshape(1, _H), dec_W2,
                 dec_b2.reshape(1, _D_OUT)],
                [128, 128, _H, 16, 16], [_D_OUT])
    return y


# speedup vs baseline: 8.0646x; 1.2022x over previous
import functools

import jax
import jax.numpy as jnp
from jax import lax
from jax.experimental import pallas as pl
from jax.experimental.pallas import tpu as pltpu
from jax.experimental.pallas import tpu_sc as plsc

_N = 10000
_E = 320000
_D_IN = 128
_H = 256
_D_OUT = 128
_DEPTH = 3

_NC = 2
_NS = 16
_B = 128

_EPAD = 327680
_EROWS = _EPAD // _B
_RPW_FEAT = _EROWS // _NS
_RPW_HIST = _EROWS // (_NC * _NS)
_NTAB = 10240
_TPW = _NTAB // _NS
_CHUNK = 16

_f32 = jnp.float32
_i32 = jnp.int32

@functools.cache
def _mesh():
    return plsc.VectorSubcoreMesh(core_axis_name="c", subcore_axis_name="s",
                                  num_cores=_NC, num_subcores=_NS)


def _fill(buf, nrows, ncols, value):
    @pl.loop(0, nrows)
    def _(r):
        @pl.loop(0, ncols // 16)
        def _(j):
            buf[r, pl.ds(j * 16, 16)] = jnp.full((16,), value, _f32)


def _init_table_slice(zero_src, table, base, nrows):
    full, rem = nrows // _B, nrows % _B
    for k in range(full):
        pltpu.sync_copy(zero_src, table.at[pl.ds(base + k * _B, _B)])
    if rem:
        pltpu.sync_copy(zero_src.at[pl.ds(0, rem)],
                        table.at[pl.ds(base + full * _B, rem)])


@functools.cache
def _sc_degree_kernel():
    return pl.kernel(
        _sc_degree_body,
        out_type=(jax.ShapeDtypeStruct((_NTAB, 16), _f32),
                  jax.ShapeDtypeStruct((_NTAB, 16), _f32)),
        mesh=_mesh(),
        scratch_types=[
            pltpu.VMEM((_RPW_HIST, _B), _i32),
            pltpu.VMEM((_B, 16), _f32),
            pltpu.VMEM_SHARED((_NTAB, 16), _f32),
        ],
    )


def _sc_degree_body(dst_hbm, out0, out1, dstv, buf, table):
    c = lax.axis_index("c")
    s = lax.axis_index("s")
    w = c * _NS + s

    _fill(buf, _B, 16, 0.0)
    _init_table_slice(buf, table, s * _TPW, _TPW)
    _fill(buf, _B, 16, 1.0)
    plsc.subcore_barrier()

    pltpu.sync_copy(dst_hbm.at[pl.ds(w * _RPW_HIST, _RPW_HIST)], dstv)

    @pl.loop(0, _RPW_HIST)
    def _(b):
        pltpu.sync_copy(buf, table.at[dstv.at[b]], add=True)

    plsc.subcore_barrier()

    @pl.when(c == 0)
    def _():
        pltpu.sync_copy(table.at[pl.ds(s * _TPW, _TPW)],
                        out0.at[pl.ds(s * _TPW, _TPW)])

    @pl.when(c == 1)
    def _():
        pltpu.sync_copy(table.at[pl.ds(s * _TPW, _TPW)],
                        out1.at[pl.ds(s * _TPW, _TPW)])


@functools.cache
def _sc_aggregate_kernel():
    return pl.kernel(
        _sc_aggregate_body,
        out_type=(jax.ShapeDtypeStruct((_NTAB, 128), _f32),
                  jax.ShapeDtypeStruct((_NTAB, 128), _f32)),
        mesh=_mesh(),
        scratch_types=[
            pltpu.VMEM((_CHUNK, _B), _i32),
            pltpu.VMEM((_CHUNK, _B), _i32),
            pltpu.VMEM((_B, 128), _f32),
            pltpu.VMEM((_B, 128), _f32),
            pltpu.VMEM_SHARED((_NTAB, 128), _f32),
            pltpu.SemaphoreType.DMA,
            pltpu.SemaphoreType.DMA,
        ],
    )


def _sc_aggregate_body(hlo_hbm, hhi_hbm, src_hbm, dst_hbm, alo, ahi,
                       srcv, dstv, rows0, rows1, table, sem0, sem1):
    c = lax.axis_index("c")
    s = lax.axis_index("s")

    _fill(rows0, _B, 128, 0.0)
    _init_table_slice(rows0, table, s * _TPW, _TPW)
    plsc.subcore_barrier()

    ring = (rows0, sem0), (rows1, sem1)

    def _gather(b, buf, sem):
        @pl.when(c == 0)
        def _():
            pltpu.async_copy(hlo_hbm.at[srcv.at[b]], buf, sem)

        @pl.when(c == 1)
        def _():
            pltpu.async_copy(hhi_hbm.at[srcv.at[b]], buf, sem)

    def _drain(buf, sem):
        pltpu.make_async_copy(hlo_hbm.at[srcv.at[0]], buf, sem).wait()

    @pl.loop(0, _RPW_FEAT // _CHUNK)
    def _(ck):
        base = s * _RPW_FEAT + ck * _CHUNK
        pltpu.sync_copy(src_hbm.at[pl.ds(base, _CHUNK)], srcv)
        pltpu.sync_copy(dst_hbm.at[pl.ds(base, _CHUNK)], dstv)

        _gather(0, *ring[0])
        for b in range(_CHUNK):
            buf, sem = ring[b % 2]
            if b + 1 < _CHUNK:
                _gather(b + 1, *ring[(b + 1) % 2])
            _drain(buf, sem)
            pltpu.sync_copy(buf, table.at[dstv.at[b]], add=True)

    plsc.subcore_barrier()

    @pl.when(c == 0)
    def _():
        pltpu.sync_copy(table.at[pl.ds(s * _TPW, _TPW)],
                        alo.at[pl.ds(s * _TPW, _TPW)])

    @pl.when(c == 1)
    def _():
        pltpu.sync_copy(table.at[pl.ds(s * _TPW, _TPW)],
                        ahi.at[pl.ds(s * _TPW, _TPW)])


_BLK = 1000


def _dot(a, b):
    return lax.dot_general(a, b, (((1,), (0,)), ((), ())),
                           precision=lax.Precision.HIGHEST,
                           preferred_element_type=_f32)


def _dinv_block(h0_ref, h1_ref):
    deg = 1.0 + h0_ref[...][:, 0:1] + h1_ref[...][:, 0:1]
    return lax.rsqrt(deg)


def _encoder_body(x_ref, w1_ref, b1_ref, w2_ref, b2_ref, o_ref):
    h = jnp.maximum(_dot(x_ref[...], w1_ref[...]) + b1_ref[...], 0.0)
    o_ref[...] = _dot(h, w2_ref[...]) + b2_ref[...]


def _prescale_body(h_ref, h0_ref, h1_ref, lo_ref, hi_ref):
    dinv = _dinv_block(h0_ref, h1_ref)
    hs = dinv * h_ref[...]
    lo_ref[...] = hs[:, :128]
    hi_ref[...] = hs[:, 128:]


def _conv_mid_body(alo_ref, ahi_ref, h_ref, h0_ref, h1_ref, w_ref, b_ref,
                   oh_ref, lo_ref, hi_ref):
    dinv = _dinv_block(h0_ref, h1_ref)
    agg = jnp.concatenate([alo_ref[...], ahi_ref[...]], axis=1)
    a = dinv * agg + (dinv * dinv) * h_ref[...]
    y = _dot(a, w_ref[...]) + b_ref[...]
    hn = h_ref[...] + jnp.maximum(y, 0.0)
    oh_ref[...] = hn
    hs = dinv * hn
    lo_ref[...] = hs[:, :128]
    hi_ref[...] = hs[:, 128:]


def _conv_last_body(alo_ref, ahi_ref, h_ref, h0_ref, h1_ref, w_ref, b_ref,
                    dw1_ref, db1_ref, dw2_ref, db2_ref, y_ref):
    dinv = _dinv_block(h0_ref, h1_ref)
    agg = jnp.concatenate([alo_ref[...], ahi_ref[...]], axis=1)
    a = dinv * agg + (dinv * dinv) * h_ref[...]
    y = _dot(a, w_ref[...]) + b_ref[...]
    hn = h_ref[...] + jnp.maximum(y, 0.0)
    y1 = jnp.maximum(_dot(hn, dw1_ref[...]) + db1_ref[...], 0.0)
    y_ref[...] = _dot(y1, dw2_ref[...]) + db2_ref[...]


def _row_spec(cols):
    return pl.BlockSpec((_BLK, cols), lambda i: (i, 0))


def _full_spec(shape):
    nd = len(shape)
    return pl.BlockSpec(shape, lambda i: (0,) * nd)


def _tc_call(body, in_arrays, row_cols_in, row_cols_out):
    in_specs = [_row_spec(cc) for cc in row_cols_in]
    in_specs += [_full_spec(a.shape) for a in in_arrays[len(row_cols_in):]]
    out_specs = tuple(_row_spec(cc) for cc in row_cols_out)
    out_shape = tuple(jax.ShapeDtypeStruct((_N, cc), _f32)
                      for cc in row_cols_out)
    if len(out_shape) == 1:
        out_shape, out_specs = out_shape[0], out_specs[0]
    return pl.pallas_call(
        body,
        grid=(_N // _BLK,),
        in_specs=in_specs,
        out_specs=out_specs,
        out_shape=out_shape,
    )(*in_arrays)


def kernel(x, edge_index, enc_W1, enc_b1, enc_W2, enc_b2, conv_W, conv_b,
           dec_W1, dec_b1, dec_W2, dec_b2):
    pad = _EPAD - _E
    src = jnp.concatenate([edge_index[0], jnp.zeros((pad,), _i32)])
    dst = jnp.concatenate([edge_index[1], jnp.full((pad,), _N, _i32)])
    src2d = src.reshape(_EROWS, _B)
    dst2d = dst.reshape(_EROWS, _B)

    hist0, hist1 = _sc_degree_kernel()(dst2d)
    h0 = hist0[:_N]
    h1 = hist1[:_N]

    h = _tc_call(_encoder_body,
                 [x, enc_W1, enc_b1.reshape(1, _H), enc_W2,
                  enc_b2.reshape(1, _H)],
                 [_D_IN], [_H])

    hlo, hhi = _tc_call(_prescale_body, [h, h0, h1],
                        [_H, 16, 16], [128, 128])

    for l in range(_DEPTH):
        alo, ahi = _sc_aggregate_kernel()(hlo, hhi, src2d, dst2d)
        alo = alo[:_N]
        ahi = ahi[:_N]
        if l < _DEPTH - 1:
            h, hlo, hhi = _tc_call(
                _conv_mid_body,
                [alo, ahi, h, h0, h1, conv_W[l], conv_b[l].reshape(1, _H)],
                [128, 128, _H, 16, 16], [_H, 128, 128])
        else:
            y = _tc_call(
                _conv_last_body,
                [alo, ahi, h, h0, h1, conv_W[l], conv_b[l].reshape(1, _H),
                 dec_W1, dec_b1.reshape(1, _H), dec_W2,
                 dec_b2.reshape(1, _D_OUT)],
                [128, 128, _H, 16, 16], [_D_OUT])
    return y
